# baseline (speedup 1.0000x reference)
import functools

import jax
import jax.numpy as jnp
from jax import lax
from jax.experimental import pallas as pl
from jax.experimental.pallas import tpu as pltpu
from jax.experimental.pallas import tpu_sc as plsc

NUM_EMB = 1000000
D = 64
B = 16384
L = 16
NC, NS = 2, 16
NW = NC * NS
BPW = B // NW
GCH = 128
NG = BPW // GCH


def _rsqrt(x):
    i = plsc.bitcast(x, jnp.int32)
    i = jnp.int32(0x5F3759DF) - (i >> 1)
    y = plsc.bitcast(i, jnp.float32)
    for _ in range(3):
        y = y * (1.5 - 0.5 * x * y * y)
    return y


@functools.cache
def _build():
    @functools.partial(
        pl.kernel,
        out_type=jax.ShapeDtypeStruct((B,), jnp.float32),
        mesh=plsc.VectorSubcoreMesh(core_axis_name="c", subcore_axis_name="s"),
        compiler_params=pltpu.CompilerParams(
            needs_layout_passes=False, use_tc_tiling_on_sc=False),
        scratch_types=[
            pltpu.VMEM((NG, GCH), jnp.int32),
            pltpu.VMEM((NG, GCH), jnp.int32),
            pltpu.VMEM((BPW, D), jnp.float32),
            pltpu.VMEM((BPW, D), jnp.float32),
            pltpu.VMEM((L, L + 1), jnp.float32),
            pltpu.VMEM((L, L + 1), jnp.float32),
            pltpu.VMEM((L, L + 1), jnp.float32),
            pltpu.VMEM((BPW,), jnp.float32),
            pltpu.SemaphoreType.DMA,
        ],
    )
    def _cos_sim_sc(w1_hbm, w2_hbm, table_hbm, out_hbm,
                    idx1_v, idx2_v, rows1_v, rows2_v,
                    dot_v, n1_v, n2_v, out_v, sem):
        _body(w1_hbm, w2_hbm, table_hbm, out_hbm,
              idx1_v, idx2_v, rows1_v, rows2_v,
              dot_v, n1_v, n2_v, out_v, sem)

    return _cos_sim_sc


def _body(w1_hbm, w2_hbm, table_hbm, out_hbm,
          idx1_v, idx2_v, rows1_v, rows2_v,
          dot_v, n1_v, n2_v, out_v, sem):
    wid = lax.axis_index("s") * NC + lax.axis_index("c")
    base = wid * BPW

    pltpu.sync_copy(w1_hbm.at[wid], idx1_v)
    pltpu.sync_copy(w2_hbm.at[wid], idx2_v)

    cps = []
    for j in range(NG):
        cps.append(pltpu.async_copy(
            table_hbm.at[idx1_v.at[j]], rows1_v.at[pl.ds(j * GCH, GCH)], sem))
        cps.append(pltpu.async_copy(
            table_hbm.at[idx2_v.at[j]], rows2_v.at[pl.ds(j * GCH, GCH)], sem))
    for cp in cps:
        cp.wait()

    iota = lax.iota(jnp.int32, L)

    def group_body(g, carry):
        for i in range(L):
            p = g * L + i
            a0 = rows1_v[p, pl.ds(0 * L, L)]
            a1 = rows1_v[p, pl.ds(1 * L, L)]
            a2 = rows1_v[p, pl.ds(2 * L, L)]
            a3 = rows1_v[p, pl.ds(3 * L, L)]
            b0 = rows2_v[p, pl.ds(0 * L, L)]
            b1 = rows2_v[p, pl.ds(1 * L, L)]
            b2 = rows2_v[p, pl.ds(2 * L, L)]
            b3 = rows2_v[p, pl.ds(3 * L, L)]
            dot_v[i, pl.ds(0, L)] = a0 * b0 + a1 * b1 + a2 * b2 + a3 * b3
            n1_v[i, pl.ds(0, L)] = a0 * a0 + a1 * a1 + a2 * a2 + a3 * a3
            n2_v[i, pl.ds(0, L)] = b0 * b0 + b1 * b1 + b2 * b2 + b3 * b3
        acc_d = plsc.load_gather(dot_v, [iota, jnp.full((L,), 0, jnp.int32)])
        acc_1 = plsc.load_gather(n1_v, [iota, jnp.full((L,), 0, jnp.int32)])
        acc_2 = plsc.load_gather(n2_v, [iota, jnp.full((L,), 0, jnp.int32)])
        for j in range(1, L):
            cj = jnp.full((L,), j, jnp.int32)
            acc_d = acc_d + plsc.load_gather(dot_v, [iota, cj])
            acc_1 = acc_1 + plsc.load_gather(n1_v, [iota, cj])
            acc_2 = acc_2 + plsc.load_gather(n2_v, [iota, cj])
        s1 = jnp.maximum(acc_1, 1e-16)
        s2 = jnp.maximum(acc_2, 1e-16)
        out_v[pl.ds(g * L, L)] = acc_d * _rsqrt(s1) * _rsqrt(s2)
        return carry

    lax.fori_loop(0, BPW // L, group_body, 0)

    pltpu.sync_copy(out_v, out_hbm.at[pl.ds(base, BPW)])


def kernel(words1, words2, table):
    w1 = words1.astype(jnp.int32).reshape(NW, NG, GCH)
    w2 = words2.astype(jnp.int32).reshape(NW, NG, GCH)
    return _build()(w1, w2, table)
